# Initial kernel scaffold; baseline (speedup 1.0000x reference)
import jax
import jax.numpy as jnp
from jax.experimental import pallas as pl


def kernel(x, edge_index, Wl_list, Wr_list, b_list):
    raise NotImplementedError("write your pallas kernel here")


# speedup vs baseline: 4.7666x; 4.7666x over previous
import functools

import jax
import jax.numpy as jnp
from jax import lax
from jax.experimental import pallas as pl
from jax.experimental.pallas import tpu as pltpu
from jax.experimental.pallas import tpu_sc as plsc

NC = 2
NS = 16
NW = NC * NS
W = 80


@functools.partial(jax.jit, static_argnames=("n", "e", "d"))
def _sc_segment_sum(p, src2, dst2, zeros_nd, *, n, e, d):
    epw = e // NW
    nwin = epw // W
    stride, chunk = 624, 640

    mesh = plsc.VectorSubcoreMesh(core_axis_name="c", subcore_axis_name="s")

    @functools.partial(
        pl.kernel,
        out_type=jax.ShapeDtypeStruct((NC, n, d), jnp.float32),
        mesh=mesh,
        scratch_types=[
            pltpu.VMEM_SHARED((n, d), jnp.float32),
            pltpu.VMEM((W,), jnp.int32),
            pltpu.VMEM((W,), jnp.int32),
            pltpu.VMEM((W, d), jnp.float32),
            pltpu.SemaphoreType.DMA,
        ],
        compiler_params=pltpu.CompilerParams(use_tc_tiling_on_sc=False),
    )
    def k(p_hbm, src_hbm, dst_hbm, z_hbm, out_hbm, acc_sh, srcv, dstv, rows, sem):
        c = lax.axis_index("c")
        s = lax.axis_index("s")
        wid = s * NC + c
        base = wid * epw

        pltpu.sync_copy(z_hbm.at[pl.ds(s * stride, chunk)],
                        acc_sh.at[pl.ds(s * stride, chunk)])
        plsc.subcore_barrier()

        def body(j, carry):
            pltpu.sync_copy(src_hbm.at[pl.ds(base + j * W, W)], srcv)
            pltpu.sync_copy(dst_hbm.at[pl.ds(base + j * W, W)], dstv)
            pltpu.async_copy(p_hbm.at[srcv], rows, sem).wait()
            pltpu.sync_copy(rows, acc_sh.at[dstv], add=True)
            return carry

        lax.fori_loop(0, nwin, body, 0, unroll=False)

        plsc.subcore_barrier()
        pltpu.sync_copy(acc_sh.at[pl.ds(s * stride, chunk)],
                        out_hbm.at[c].at[pl.ds(s * stride, chunk)])

    return k(p, src2, dst2, zeros_nd)


@functools.partial(jax.jit, static_argnames=("n_pad", "e"))
def _sc_degree(dst2, zeros_n, *, n_pad, e):
    epw = e // NW
    nwin = epw // W
    stride = chunk = n_pad // NS

    mesh = plsc.VectorSubcoreMesh(core_axis_name="c", subcore_axis_name="s")

    @functools.partial(
        pl.kernel,
        out_type=jax.ShapeDtypeStruct((NC, n_pad), jnp.float32),
        mesh=mesh,
        scratch_types=[
            pltpu.VMEM_SHARED((n_pad,), jnp.float32),
            pltpu.VMEM((W,), jnp.int32),
            pltpu.VMEM((W,), jnp.float32),
        ],
        compiler_params=pltpu.CompilerParams(use_tc_tiling_on_sc=False),
    )
    def k(dst_hbm, z_hbm, out_hbm, acc_sh, dstv, ones_v):
        c = lax.axis_index("c")
        s = lax.axis_index("s")
        wid = s * NC + c
        base = wid * epw

        one = jnp.ones((16,), jnp.float32)
        for i in range(W // 16):
            ones_v[pl.ds(i * 16, 16)] = one

        pltpu.sync_copy(z_hbm.at[pl.ds(s * stride, chunk)],
                        acc_sh.at[pl.ds(s * stride, chunk)])
        plsc.subcore_barrier()

        def body(j, carry):
            pltpu.sync_copy(dst_hbm.at[pl.ds(base + j * W, W)], dstv)
            pltpu.sync_copy(ones_v, acc_sh.at[dstv], add=True)
            return carry

        lax.fori_loop(0, nwin, body, 0, unroll=False)

        plsc.subcore_barrier()
        pltpu.sync_copy(acc_sh.at[pl.ds(s * stride, chunk)],
                        out_hbm.at[c].at[pl.ds(s * stride, chunk)])

    return k(dst2, zeros_n)


_BLK = 1000


def _full(shape):
    return pl.BlockSpec(shape, lambda i: (0,) * len(shape))


def _rows(shape):
    return pl.BlockSpec(shape, lambda i: (i,) + (0,) * (len(shape) - 1))


def _tc_pre(x, wl, wr, b, *, n):
    din, dout = wl.shape
    _, dout_r = wr.shape

    def body(x_ref, wl_ref, wr_ref, b_ref, p_ref, r_ref):
        xb = x_ref[...]
        p_ref[...] = jnp.dot(xb, wl_ref[...], preferred_element_type=jnp.float32)
        r_ref[...] = jnp.dot(xb, wr_ref[...], preferred_element_type=jnp.float32) + b_ref[...]

    return pl.pallas_call(
        body,
        grid=(n // _BLK,),
        in_specs=[_rows((_BLK, din)), _full((din, dout)), _full((din, dout_r)), _full((1, dout_r))],
        out_specs=[_rows((_BLK, dout)), _rows((_BLK, dout_r))],
        out_shape=[
            jax.ShapeDtypeStruct((n, dout), jnp.float32),
            jax.ShapeDtypeStruct((n, dout_r), jnp.float32),
        ],
    )(x, wl, wr, b)


def _tc_combine_project(sp, degp, r_prev, wl, wr, b, *, n):
    din, dout = wl.shape
    _, dout_r = wr.shape

    def body(s_ref, deg_ref, rp_ref, wl_ref, wr_ref, b_ref, p_ref, r_ref):
        ssum = s_ref[0] + s_ref[1]
        deg = jnp.maximum(deg_ref[0] + deg_ref[1], 1.0)
        h = jnp.maximum(ssum / deg + rp_ref[...], 0.0)
        p_ref[...] = jnp.dot(h, wl_ref[...], preferred_element_type=jnp.float32)
        r_ref[...] = jnp.dot(h, wr_ref[...], preferred_element_type=jnp.float32) + b_ref[...]

    return pl.pallas_call(
        body,
        grid=(n // _BLK,),
        in_specs=[
            pl.BlockSpec((NC, _BLK, din), lambda i: (0, i, 0)),
            pl.BlockSpec((NC, _BLK, 1), lambda i: (0, i, 0)),
            _rows((_BLK, din)),
            _full((din, dout)),
            _full((din, dout_r)),
            _full((1, dout_r)),
        ],
        out_specs=[_rows((_BLK, dout)), _rows((_BLK, dout_r))],
        out_shape=[
            jax.ShapeDtypeStruct((n, dout), jnp.float32),
            jax.ShapeDtypeStruct((n, dout_r), jnp.float32),
        ],
    )(sp, degp, r_prev, wl, wr, b)


def _tc_final(sp, degp, r_prev, *, n, dout):
    dpad = sp.shape[-1]

    def body(s_ref, deg_ref, rp_ref, o_ref):
        ssum = (s_ref[0] + s_ref[1])[:, :dout]
        deg = jnp.maximum(deg_ref[0] + deg_ref[1], 1.0)
        o_ref[...] = ssum / deg + rp_ref[...]

    return pl.pallas_call(
        body,
        grid=(n // _BLK,),
        in_specs=[
            pl.BlockSpec((NC, _BLK, dpad), lambda i: (0, i, 0)),
            pl.BlockSpec((NC, _BLK, 1), lambda i: (0, i, 0)),
            _rows((_BLK, dout)),
        ],
        out_specs=_rows((_BLK, dout)),
        out_shape=jax.ShapeDtypeStruct((n, dout), jnp.float32),
    )(sp, degp, r_prev)


def kernel(x, edge_index, Wl_list, Wr_list, b_list):
    n, din = x.shape
    e = edge_index.shape[1]
    n_layers = len(Wl_list) - 1

    src2 = edge_index[0]
    dst2 = edge_index[1]
    n_pad = NS * 640
    zeros_n = jnp.zeros((n_pad,), jnp.float32)

    degp = _sc_degree(dst2, zeros_n, n_pad=n_pad, e=e)
    degp3 = degp[:, :n, None]

    d_out = Wl_list[n_layers].shape[1]
    d_pad = 32
    wl_last = jnp.zeros((Wl_list[n_layers].shape[0], d_pad), jnp.float32)
    wl_last = wl_last.at[:, :d_out].set(Wl_list[n_layers])

    wls = list(Wl_list[:n_layers]) + [wl_last]
    wrs = list(Wr_list)
    bs = [b.reshape(1, -1) for b in b_list]

    p, r = _tc_pre(x, wls[0], wrs[0], bs[0], n=n)
    for i in range(n_layers):
        d = p.shape[1]
        zeros_nd = jnp.zeros((n, d), jnp.float32)
        sp = _sc_segment_sum(p, src2, dst2, zeros_nd, n=n, e=e, d=d)
        p, r = _tc_combine_project(sp, degp3, r, wls[i + 1], wrs[i + 1], bs[i + 1], n=n)

    d = p.shape[1]
    zeros_nd = jnp.zeros((n, d), jnp.float32)
    sp = _sc_segment_sum(p, src2, dst2, zeros_nd, n=n, e=e, d=d)
    return _tc_final(sp, degp3, r, n=n, dout=d_out)


# speedup vs baseline: 16.0018x; 3.3571x over previous
import functools

import jax
import jax.numpy as jnp
from jax import lax
from jax.experimental import pallas as pl
from jax.experimental.pallas import tpu as pltpu
from jax.experimental.pallas import tpu_sc as plsc

NC = 2
NS = 16
NW = NC * NS
W = 80


NBUF = 5


@functools.partial(jax.jit, static_argnames=("n", "e", "d", "with_deg", "n_pad"))
def _sc_segment_sum(p, src2, dst2, zeros_nd, zeros_np, *, n, e, d,
                    with_deg=False, n_pad=0):
    epw = e // NW
    nwin = epw // W
    stride, chunk = 624, 640

    mesh = plsc.VectorSubcoreMesh(core_axis_name="c", subcore_axis_name="s")

    out_types = [jax.ShapeDtypeStruct((NC, n, d), jnp.float32)]
    scratch = [
        pltpu.VMEM_SHARED((n, d), jnp.float32),
        pltpu.VMEM((epw,), jnp.int32),
        pltpu.VMEM((epw,), jnp.int32),
        [pltpu.VMEM((W, d), jnp.float32) for _ in range(NBUF)],
        [pltpu.SemaphoreType.DMA for _ in range(NBUF)],
    ]
    if with_deg:
        out_types.append(jax.ShapeDtypeStruct((NC, n_pad), jnp.float32))
        scratch += [
            pltpu.VMEM_SHARED((n_pad,), jnp.float32),
            pltpu.VMEM((W,), jnp.float32),
        ]

    @functools.partial(
        pl.kernel,
        out_type=tuple(out_types),
        mesh=mesh,
        scratch_types=scratch,
        compiler_params=pltpu.CompilerParams(use_tc_tiling_on_sc=False),
    )
    def k(p_hbm, src_hbm, dst_hbm, z_hbm, zp_hbm, *refs):
        if with_deg:
            (out_hbm, deg_hbm, acc_sh, srcall, dstall, rows, sems,
             dacc_sh, ones_v) = refs
        else:
            out_hbm, acc_sh, srcall, dstall, rows, sems = refs

        c = lax.axis_index("c")
        s = lax.axis_index("s")
        wid = s * NC + c
        base = wid * epw

        pltpu.sync_copy(z_hbm.at[pl.ds(s * stride, chunk)],
                        acc_sh.at[pl.ds(s * stride, chunk)])
        if with_deg:
            dchunk = n_pad // NS
            pltpu.sync_copy(zp_hbm.at[pl.ds(s * dchunk, dchunk)],
                            dacc_sh.at[pl.ds(s * dchunk, dchunk)])
            one = jnp.ones((16,), jnp.float32)
            for i in range(W // 16):
                ones_v[pl.ds(i * 16, 16)] = one
        pltpu.sync_copy(src_hbm.at[pl.ds(base, epw)], srcall)
        pltpu.sync_copy(dst_hbm.at[pl.ds(base, epw)], dstall)
        plsc.subcore_barrier()

        for b in range(NBUF):
            pltpu.async_copy(p_hbm.at[srcall.at[pl.ds(b * W, W)]],
                             rows[b], sems[b])

        def round_(j0):
            for b in range(NBUF):
                j = j0 + b
                pltpu.make_async_copy(p_hbm.at[srcall.at[pl.ds(0, W)]],
                                      rows[b], sems[b]).wait()
                pltpu.sync_copy(rows[b],
                                acc_sh.at[dstall.at[pl.ds(j * W, W)]],
                                add=True)
                if with_deg:
                    pltpu.sync_copy(ones_v,
                                    dacc_sh.at[dstall.at[pl.ds(j * W, W)]],
                                    add=True)
                yield b, j

        def body(m, carry):
            j0 = m * NBUF
            for b, j in round_(j0):
                pltpu.async_copy(p_hbm.at[srcall.at[pl.ds((j + NBUF) * W, W)]],
                                 rows[b], sems[b])
            return carry

        lax.fori_loop(0, nwin // NBUF - 1, body, 0, unroll=False)
        for _ in round_(nwin - NBUF):
            pass

        plsc.subcore_barrier()
        pltpu.sync_copy(acc_sh.at[pl.ds(s * stride, chunk)],
                        out_hbm.at[c].at[pl.ds(s * stride, chunk)])
        if with_deg:
            dchunk = n_pad // NS
            pltpu.sync_copy(dacc_sh.at[pl.ds(s * dchunk, dchunk)],
                            deg_hbm.at[c].at[pl.ds(s * dchunk, dchunk)])

    return k(p, src2, dst2, zeros_nd, zeros_np)


_BLK = 1000


def _full(shape):
    return pl.BlockSpec(shape, lambda i: (0,) * len(shape))


def _rows(shape):
    return pl.BlockSpec(shape, lambda i: (i,) + (0,) * (len(shape) - 1))


def _tc_pre(x, wl, wr, b, *, n):
    din, dout = wl.shape
    _, dout_r = wr.shape

    def body(x_ref, wl_ref, wr_ref, b_ref, p_ref, r_ref):
        xb = x_ref[...]
        p_ref[...] = jnp.dot(xb, wl_ref[...], preferred_element_type=jnp.float32)
        r_ref[...] = jnp.dot(xb, wr_ref[...], preferred_element_type=jnp.float32) + b_ref[...]

    return pl.pallas_call(
        body,
        grid=(n // _BLK,),
        in_specs=[_rows((_BLK, din)), _full((din, dout)), _full((din, dout_r)), _full((1, dout_r))],
        out_specs=[_rows((_BLK, dout)), _rows((_BLK, dout_r))],
        out_shape=[
            jax.ShapeDtypeStruct((n, dout), jnp.float32),
            jax.ShapeDtypeStruct((n, dout_r), jnp.float32),
        ],
    )(x, wl, wr, b)


def _tc_combine_project(sp, degp, r_prev, wl, wr, b, *, n):
    din, dout = wl.shape
    _, dout_r = wr.shape

    def body(s_ref, deg_ref, rp_ref, wl_ref, wr_ref, b_ref, p_ref, r_ref):
        ssum = s_ref[0] + s_ref[1]
        deg = jnp.maximum(deg_ref[0] + deg_ref[1], 1.0)
        h = jnp.maximum(ssum / deg + rp_ref[...], 0.0)
        p_ref[...] = jnp.dot(h, wl_ref[...], preferred_element_type=jnp.float32)
        r_ref[...] = jnp.dot(h, wr_ref[...], preferred_element_type=jnp.float32) + b_ref[...]

    return pl.pallas_call(
        body,
        grid=(n // _BLK,),
        in_specs=[
            pl.BlockSpec((NC, _BLK, din), lambda i: (0, i, 0)),
            pl.BlockSpec((NC, _BLK, 1), lambda i: (0, i, 0)),
            _rows((_BLK, din)),
            _full((din, dout)),
            _full((din, dout_r)),
            _full((1, dout_r)),
        ],
        out_specs=[_rows((_BLK, dout)), _rows((_BLK, dout_r))],
        out_shape=[
            jax.ShapeDtypeStruct((n, dout), jnp.float32),
            jax.ShapeDtypeStruct((n, dout_r), jnp.float32),
        ],
    )(sp, degp, r_prev, wl, wr, b)


def _tc_final(sp, degp, r_prev, *, n, dout):
    dpad = sp.shape[-1]

    def body(s_ref, deg_ref, rp_ref, o_ref):
        ssum = (s_ref[0] + s_ref[1])[:, :dout]
        deg = jnp.maximum(deg_ref[0] + deg_ref[1], 1.0)
        o_ref[...] = ssum / deg + rp_ref[...]

    return pl.pallas_call(
        body,
        grid=(n // _BLK,),
        in_specs=[
            pl.BlockSpec((NC, _BLK, dpad), lambda i: (0, i, 0)),
            pl.BlockSpec((NC, _BLK, 1), lambda i: (0, i, 0)),
            _rows((_BLK, dout)),
        ],
        out_specs=_rows((_BLK, dout)),
        out_shape=jax.ShapeDtypeStruct((n, dout), jnp.float32),
    )(sp, degp, r_prev)


def kernel(x, edge_index, Wl_list, Wr_list, b_list):
    n, din = x.shape
    e = edge_index.shape[1]
    n_layers = len(Wl_list) - 1

    src2 = edge_index[0]
    dst2 = edge_index[1]
    n_pad = NS * 640
    zeros_np = jnp.zeros((n_pad,), jnp.float32)

    d_out = Wl_list[n_layers].shape[1]
    d_pad = 32
    wl_last = jnp.zeros((Wl_list[n_layers].shape[0], d_pad), jnp.float32)
    wl_last = wl_last.at[:, :d_out].set(Wl_list[n_layers])

    wls = list(Wl_list[:n_layers]) + [wl_last]
    wrs = list(Wr_list)
    bs = [b.reshape(1, -1) for b in b_list]

    p, r = _tc_pre(x, wls[0], wrs[0], bs[0], n=n)
    zeros_nd = jnp.zeros((n, p.shape[1]), jnp.float32)

    sp, degp = _sc_segment_sum(p, src2, dst2, zeros_nd, zeros_np,
                               n=n, e=e, d=p.shape[1],
                               with_deg=True, n_pad=n_pad)
    degp3 = degp[:, :n, None]
    p, r = _tc_combine_project(sp, degp3, r, wls[1], wrs[1], bs[1], n=n)

    for i in range(1, n_layers):
        d = p.shape[1]
        zeros_d = jnp.zeros((n, d), jnp.float32)
        (sp,) = _sc_segment_sum(p, src2, dst2, zeros_d, zeros_np,
                                n=n, e=e, d=d)
        p, r = _tc_combine_project(sp, degp3, r, wls[i + 1], wrs[i + 1], bs[i + 1], n=n)

    d = p.shape[1]
    zeros_d = jnp.zeros((n, d), jnp.float32)
    (sp,) = _sc_segment_sum(p, src2, dst2, zeros_d, zeros_np, n=n, e=e, d=d)
    return _tc_final(sp, degp3, r, n=n, dout=d_out)
